# baseline (speedup 1.0000x reference)
import jax
import jax.numpy as jnp
from jax import lax
from jax.experimental import pallas as pl
from jax.experimental.pallas import tpu as pltpu
from jax.experimental.pallas import tpu_sc as plsc

D_VOCAB = 100000
HALF = D_VOCAB // 2
N_ROWS = 1024
NC = 2
NS = 16
L = 16
NW = NC * NS
ROWS_PER_W = N_ROWS // NW


def _body(x_hbm, out_hbm, idx_v, buf0, buf1, sem0, sem1):
    wid = lax.axis_index("s") * NC + lax.axis_index("c")
    base = wid * ROWS_PER_W

    pltpu.sync_copy(x_hbm.at[pl.ds(base * 2, 2 * ROWS_PER_W)], idx_v)

    zeros16 = jnp.zeros((L,), jnp.float32)
    ones16 = jnp.ones((L,), jnp.float32)
    iota16 = lax.iota(jnp.int32, L)

    bufs = (buf0, buf1)
    sems = (sem0, sem1)

    def _zero(i, carry):
        buf0[pl.ds(i * L, L)] = zeros16
        buf1[pl.ds(i * L, L)] = zeros16
        return carry

    lax.fori_loop(0, HALF // L, _zero, 0)

    handles = [None, None]
    prev = [None, None]
    for r in range(ROWS_PER_W):
        c, k = divmod(r, 8)
        if k == 0:
            chunk = idx_v[pl.ds(c * L, L)]
            in0 = chunk < HALF
        m0 = iota16 == (2 * k)
        m1 = iota16 == (2 * k + 1)
        for h in range(2):
            inh = in0 if h == 0 else jnp.logical_not(in0)
            local = jnp.clip(chunk - h * HALF, 0, HALF - 1)
            if handles[h] is not None:
                handles[h].wait()
                plsc.store_scatter(bufs[h], [prev[h][0]], zeros16,
                                   mask=prev[h][1])
            plsc.addupdate_scatter(bufs[h], [local], ones16, mask=m0 & inh)
            plsc.addupdate_scatter(bufs[h], [local], ones16, mask=m1 & inh)
            handles[h] = pltpu.async_copy(
                bufs[h], out_hbm.at[(base + r) * 2 + h], sems[h])
            prev[h] = (local, (m0 | m1) & inh)
    handles[0].wait()
    handles[1].wait()


@jax.jit
def _embed(x_flat):
    mesh = plsc.VectorSubcoreMesh(
        core_axis_name="c", subcore_axis_name="s", num_cores=NC,
        num_subcores=NS)
    f = pl.kernel(
        _body,
        out_type=jax.ShapeDtypeStruct((2 * N_ROWS, HALF), jnp.float32),
        mesh=mesh,
        scratch_types=[
            pltpu.VMEM((2 * ROWS_PER_W,), jnp.int32),
            pltpu.VMEM((HALF,), jnp.float32),
            pltpu.VMEM((HALF,), jnp.float32),
            pltpu.SemaphoreType.DMA,
            pltpu.SemaphoreType.DMA,
        ],
        compiler_params=pltpu.CompilerParams(needs_layout_passes=False),
    )
    return f(x_flat)


def kernel(x, W_E):
    del W_E
    out = _embed(x.reshape(-1).astype(jnp.int32))
    return out.reshape(N_ROWS, 1, D_VOCAB)
